# Initial kernel scaffold; baseline (speedup 1.0000x reference)
import jax
import jax.numpy as jnp
from jax.experimental import pallas as pl


def kernel(x, edge_index, We0, We1, Wn0, Wn1):
    raise NotImplementedError("write your pallas kernel here")


# speedup vs baseline: 2.3813x; 2.3813x over previous
import functools

import jax
import jax.numpy as jnp
from jax import lax
from jax.experimental import pallas as pl
from jax.experimental.pallas import tpu as pltpu
from jax.experimental.pallas import tpu_sc as plsc

NC = 2
NS = 16
NW = NC * NS
L = 16
BE = 96
CH = 96
ROWB = 384
NMULT = 384


def _dot_t(a, w):
    return lax.dot_general(a, w, (((1,), (1,)), ((), ())),
                           preferred_element_type=jnp.float32)


def _pre_body(x_ref, we0_ref, p_ref, q_ref):
    d = x_ref.shape[1]
    a = we0_ref[:, 0:d]
    b = we0_ref[:, d:2 * d]
    c = we0_ref[:, 2 * d:3 * d]
    xb = x_ref[...]
    p_ref[...] = _dot_t(xb, a + c)
    q_ref[...] = _dot_t(xb, b - c)


def _post_body(x_ref, s0_ref, s1_ref, xs0_ref, xs1_ref, c0_ref, c1_ref,
               we1_ref, wn0_ref, wn1_ref, o_ref):
    d = x_ref.shape[1]
    x = x_ref[...]
    s = s0_ref[...] + s1_ref[...]
    xs = xs0_ref[...] + xs1_ref[...]
    cnt = (c0_ref[...] + c1_ref[...])[:, 0:1]
    sums = xs - cnt * x + _dot_t(s, we1_ref[...])
    agg = sums / jnp.maximum(cnt, 1.0)
    u = wn0_ref[:, 0:d]
    v = wn0_ref[:, d:2 * d]
    z = _dot_t(x, u) + _dot_t(agg, v)
    t = jnp.where(z > 0, z, jnp.exp(z) - 1.0)
    o_ref[...] = x + _dot_t(t, wn1_ref[...])


@functools.lru_cache(maxsize=None)
def _build_sc_agg(npad: int, d: int, bpw: int):
    nch = npad // CH
    kmax = -(-nch // NS)
    mesh = plsc.VectorSubcoreMesh(core_axis_name="c", subcore_axis_name="s",
                                  num_cores=NC, num_subcores=NS)

    @functools.partial(
        pl.kernel,
        out_type=(
            jax.ShapeDtypeStruct((NC * npad, d), jnp.float32),
            jax.ShapeDtypeStruct((NC * npad, d), jnp.float32),
            jax.ShapeDtypeStruct((NC * npad, d), jnp.float32),
        ),
        mesh=mesh,
        scratch_types=[
            pltpu.VMEM((BE,), jnp.int32),
            pltpu.VMEM((BE,), jnp.int32),
            pltpu.VMEM((BE, d), jnp.float32),
            pltpu.VMEM((BE, d), jnp.float32),
            pltpu.VMEM_SHARED((npad, d), jnp.float32),
        ],
    )
    def sc_agg(p_hbm, q_hbm, xp_hbm, src_hbm, dst_hbm, z_hbm,
               s_out, xs_out, cnt_out,
               idx_s, idx_d, rows_a, rows_b, acc_sh):
        cid = lax.axis_index("c")
        sid = lax.axis_index("s")
        wid = sid * NC + cid

        def for_chunks(fn):
            def body(k, _):
                c = jnp.minimum(sid + NS * k, nch - 1)
                fn(c * CH)
                return 0
            lax.fori_loop(0, kmax, body, 0)

        def zero_chunk(off):
            sl = pl.ds(off, CH)
            pltpu.sync_copy(z_hbm.at[sl], acc_sh.at[sl])
        for_chunks(zero_chunk)
        plsc.subcore_barrier()

        def block_a(bi, _):
            off = (wid * bpw + bi) * BE
            pltpu.sync_copy(src_hbm.at[pl.ds(off, BE)], idx_s)
            pltpu.sync_copy(dst_hbm.at[pl.ds(off, BE)], idx_d)
            pltpu.sync_copy(p_hbm.at[idx_s], rows_a)
            pltpu.sync_copy(q_hbm.at[idx_d], rows_b)

            def elu_row(r, _):
                for cc in range(d // L):
                    sl = pl.ds(cc * L, L)
                    z = rows_a[r, sl] + rows_b[r, sl]
                    rows_a[r, sl] = jnp.where(z > 0, z, jnp.exp(z) - 1.0)
                return 0
            lax.fori_loop(0, BE, elu_row, 0)

            pltpu.sync_copy(rows_a, acc_sh.at[idx_d], add=True)
            return 0
        lax.fori_loop(0, bpw, block_a, 0)
        plsc.subcore_barrier()

        def dump_a_chunk(off):
            pltpu.sync_copy(acc_sh.at[pl.ds(off, CH)],
                            s_out.at[pl.ds(cid * npad + off, CH)])
        for_chunks(dump_a_chunk)
        plsc.subcore_barrier()

        def rezero_chunk(off):
            sl = pl.ds(off, CH)
            pltpu.sync_copy(z_hbm.at[sl], acc_sh.at[sl])
        for_chunks(rezero_chunk)
        plsc.subcore_barrier()

        def block_b(bi, _):
            off = (wid * bpw + bi) * BE
            pltpu.sync_copy(src_hbm.at[pl.ds(off, BE)], idx_s)
            pltpu.sync_copy(dst_hbm.at[pl.ds(off, BE)], idx_d)
            pltpu.sync_copy(xp_hbm.at[idx_s], rows_a)
            pltpu.sync_copy(rows_a, acc_sh.at[idx_d], add=True)
            return 0
        lax.fori_loop(0, bpw, block_b, 0)
        plsc.subcore_barrier()

        def dump_b_chunk(off):
            pltpu.sync_copy(acc_sh.at[pl.ds(off, CH)],
                            xs_out.at[pl.ds(cid * npad + off, CH)])
        for_chunks(dump_b_chunk)
        plsc.subcore_barrier()

        for_chunks(rezero_chunk)
        plsc.subcore_barrier()

        def fill_ones_row(i, _):
            for cc in range(d // L):
                rows_a[i, pl.ds(cc * L, L)] = jnp.ones((L,), jnp.float32)
            return 0
        lax.fori_loop(0, BE, fill_ones_row, 0)

        def block_c(bi, _):
            off = (wid * bpw + bi) * BE
            pltpu.sync_copy(dst_hbm.at[pl.ds(off, BE)], idx_d)
            pltpu.sync_copy(rows_a, acc_sh.at[idx_d], add=True)
            return 0
        lax.fori_loop(0, bpw, block_c, 0)
        plsc.subcore_barrier()

        def dump_c_chunk(off):
            pltpu.sync_copy(acc_sh.at[pl.ds(off, CH)],
                            cnt_out.at[pl.ds(cid * npad + off, CH)])
        for_chunks(dump_c_chunk)

    return sc_agg


def kernel(x, edge_index, We0, We1, Wn0, Wn1):
    n, d = x.shape
    e = edge_index.shape[1]
    npad = (n // NMULT + 1) * NMULT
    bpw = -(-e // (NW * BE))
    epad = bpw * NW * BE

    x_pad = jnp.pad(x, ((0, npad - n), (0, 0)))
    pad = epad - e
    src = jnp.concatenate([edge_index[0], jnp.full((pad,), n, jnp.int32)])
    dst = jnp.concatenate([edge_index[1], jnp.full((pad,), n, jnp.int32)])

    nb = npad // ROWB
    row_spec = pl.BlockSpec((ROWB, d), lambda i: (i, 0))

    p, q = pl.pallas_call(
        _pre_body,
        grid=(nb,),
        in_specs=[row_spec, pl.BlockSpec((d, 3 * d), lambda i: (0, 0))],
        out_specs=[row_spec, row_spec],
        out_shape=[jax.ShapeDtypeStruct((npad, d), jnp.float32)] * 2,
    )(x_pad, We0)

    z128 = jnp.zeros((npad, d), jnp.float32)
    s_parts, xs_parts, cnt_parts = _build_sc_agg(npad, d, bpw)(
        p, q, x_pad, src, dst, z128)
    s_parts = s_parts.reshape(NC, npad, d)
    xs_parts = xs_parts.reshape(NC, npad, d)
    cnt_parts = cnt_parts.reshape(NC, npad, d)

    out_pad = pl.pallas_call(
        _post_body,
        grid=(nb,),
        in_specs=[row_spec,
                  row_spec, row_spec, row_spec, row_spec,
                  row_spec, row_spec,
                  pl.BlockSpec((d, d), lambda i: (0, 0)),
                  pl.BlockSpec((d, 2 * d), lambda i: (0, 0)),
                  pl.BlockSpec((d, d), lambda i: (0, 0))],
        out_specs=row_spec,
        out_shape=jax.ShapeDtypeStruct((npad, d), jnp.float32),
    )(x_pad, s_parts[0], s_parts[1], xs_parts[0], xs_parts[1],
      cnt_parts[0], cnt_parts[1], We1, Wn0, Wn1)

    return out_pad[:n]


# speedup vs baseline: 2.8852x; 1.2116x over previous
import functools

import jax
import jax.numpy as jnp
from jax import lax
from jax.experimental import pallas as pl
from jax.experimental.pallas import tpu as pltpu
from jax.experimental.pallas import tpu_sc as plsc

NC = 2
NS = 16
NW = NC * NS
L = 16
BE = 96
CH = 96
ROWB = 384
NMULT = 384


def _dot_t(a, w):
    return lax.dot_general(a, w, (((1,), (1,)), ((), ())),
                           preferred_element_type=jnp.float32)


def _pre_body(x_ref, we0_ref, p_ref, q_ref):
    d = x_ref.shape[1]
    a = we0_ref[:, 0:d]
    b = we0_ref[:, d:2 * d]
    c = we0_ref[:, 2 * d:3 * d]
    xb = x_ref[...]
    p_ref[...] = _dot_t(xb, a + c)
    q_ref[...] = _dot_t(xb, b - c)


def _post_body(x_ref, s0_ref, s1_ref, xs0_ref, xs1_ref, c0_ref, c1_ref,
               we1_ref, wn0_ref, wn1_ref, o_ref):
    d = x_ref.shape[1]
    x = x_ref[...]
    s = s0_ref[...] + s1_ref[...]
    xs = xs0_ref[...] + xs1_ref[...]
    cnt = (c0_ref[...] + c1_ref[...])[:, 0:1]
    sums = xs - cnt * x + _dot_t(s, we1_ref[...])
    agg = sums / jnp.maximum(cnt, 1.0)
    u = wn0_ref[:, 0:d]
    v = wn0_ref[:, d:2 * d]
    z = _dot_t(x, u) + _dot_t(agg, v)
    t = jnp.where(z > 0, z, jnp.exp(z) - 1.0)
    o_ref[...] = x + _dot_t(t, wn1_ref[...])


@functools.lru_cache(maxsize=None)
def _build_sc_agg(npad: int, d: int, bpw: int):
    nch = npad // CH
    kmax = -(-nch // NS)
    mesh = plsc.VectorSubcoreMesh(core_axis_name="c", subcore_axis_name="s",
                                  num_cores=NC, num_subcores=NS)

    @functools.partial(
        pl.kernel,
        out_type=(
            jax.ShapeDtypeStruct((NC * npad, d), jnp.float32),
            jax.ShapeDtypeStruct((NC * npad, d), jnp.float32),
            jax.ShapeDtypeStruct((NC * npad, d), jnp.float32),
        ),
        mesh=mesh,
        scratch_types=[
            pltpu.VMEM((BE,), jnp.int32),
            pltpu.VMEM((BE,), jnp.int32),
            pltpu.VMEM((BE, d), jnp.float32),
            pltpu.VMEM((BE, d), jnp.float32),
            pltpu.VMEM_SHARED((npad, d), jnp.float32),
            pltpu.SemaphoreType.DMA,
        ],
    )
    def sc_agg(p_hbm, q_hbm, xp_hbm, src_hbm, dst_hbm, z_hbm,
               s_out, xs_out, cnt_out,
               idx_s, idx_d, rows_a, rows_b, acc_sh, sem):
        cid = lax.axis_index("c")
        sid = lax.axis_index("s")
        wid = sid * NC + cid

        def for_chunks(fn):
            def body(k, _):
                c = jnp.minimum(sid + NS * k, nch - 1)
                fn(c * CH)
                return 0
            lax.fori_loop(0, kmax, body, 0)

        def zero_chunk(off):
            sl = pl.ds(off, CH)
            pltpu.sync_copy(z_hbm.at[sl], acc_sh.at[sl])
        for_chunks(zero_chunk)
        plsc.subcore_barrier()

        def block_a(bi, _):
            off = (wid * bpw + bi) * BE
            c1 = pltpu.async_copy(src_hbm.at[pl.ds(off, BE)], idx_s, sem)
            c2 = pltpu.async_copy(dst_hbm.at[pl.ds(off, BE)], idx_d, sem)
            c1.wait()
            c2.wait()
            c3 = pltpu.async_copy(p_hbm.at[idx_s], rows_a, sem)
            c4 = pltpu.async_copy(q_hbm.at[idx_d], rows_b, sem)
            c3.wait()
            c4.wait()

            def elu_row(r, _):
                for cc in range(d // L):
                    sl = pl.ds(cc * L, L)
                    z = rows_a[r, sl] + rows_b[r, sl]
                    rows_a[r, sl] = jnp.where(z > 0, z, jnp.exp(z) - 1.0)
                return 0
            lax.fori_loop(0, BE, elu_row, 0)

            pltpu.sync_copy(rows_a, acc_sh.at[idx_d], add=True)
            return 0
        lax.fori_loop(0, bpw, block_a, 0)
        plsc.subcore_barrier()

        def dump_a_chunk(off):
            pltpu.sync_copy(acc_sh.at[pl.ds(off, CH)],
                            s_out.at[pl.ds(cid * npad + off, CH)])
        for_chunks(dump_a_chunk)
        plsc.subcore_barrier()

        def rezero_chunk(off):
            sl = pl.ds(off, CH)
            pltpu.sync_copy(z_hbm.at[sl], acc_sh.at[sl])
        for_chunks(rezero_chunk)
        plsc.subcore_barrier()

        def block_b(bi, _):
            off = (wid * bpw + bi) * BE
            c1 = pltpu.async_copy(src_hbm.at[pl.ds(off, BE)], idx_s, sem)
            c2 = pltpu.async_copy(dst_hbm.at[pl.ds(off, BE)], idx_d, sem)
            c1.wait()
            c2.wait()
            pltpu.sync_copy(xp_hbm.at[idx_s], rows_a)
            pltpu.sync_copy(rows_a, acc_sh.at[idx_d], add=True)
            return 0
        lax.fori_loop(0, bpw, block_b, 0)
        plsc.subcore_barrier()

        def dump_b_chunk(off):
            pltpu.sync_copy(acc_sh.at[pl.ds(off, CH)],
                            xs_out.at[pl.ds(cid * npad + off, CH)])
        for_chunks(dump_b_chunk)
        plsc.subcore_barrier()

        for_chunks(rezero_chunk)
        plsc.subcore_barrier()

        def fill_ones_row(i, _):
            for cc in range(d // L):
                rows_a[i, pl.ds(cc * L, L)] = jnp.ones((L,), jnp.float32)
            return 0
        lax.fori_loop(0, BE, fill_ones_row, 0)

        def block_c(bi, _):
            off = (wid * bpw + bi) * BE
            pltpu.sync_copy(dst_hbm.at[pl.ds(off, BE)], idx_d)
            pltpu.sync_copy(rows_a, acc_sh.at[idx_d], add=True)
            return 0
        lax.fori_loop(0, bpw, block_c, 0)
        plsc.subcore_barrier()

        def dump_c_chunk(off):
            pltpu.sync_copy(acc_sh.at[pl.ds(off, CH)],
                            cnt_out.at[pl.ds(cid * npad + off, CH)])
        for_chunks(dump_c_chunk)

    return sc_agg


def kernel(x, edge_index, We0, We1, Wn0, Wn1):
    n, d = x.shape
    e = edge_index.shape[1]
    npad = (n // NMULT + 1) * NMULT
    bpw = -(-e // (NW * BE))
    epad = bpw * NW * BE

    x_pad = jnp.pad(x, ((0, npad - n), (0, 0)))
    pad = epad - e
    src = jnp.concatenate([edge_index[0], jnp.full((pad,), n, jnp.int32)])
    dst = jnp.concatenate([edge_index[1], jnp.full((pad,), n, jnp.int32)])

    nb = npad // ROWB
    row_spec = pl.BlockSpec((ROWB, d), lambda i: (i, 0))

    p, q = pl.pallas_call(
        _pre_body,
        grid=(nb,),
        in_specs=[row_spec, pl.BlockSpec((d, 3 * d), lambda i: (0, 0))],
        out_specs=[row_spec, row_spec],
        out_shape=[jax.ShapeDtypeStruct((npad, d), jnp.float32)] * 2,
    )(x_pad, We0)

    z128 = jnp.zeros((npad, d), jnp.float32)
    s_parts, xs_parts, cnt_parts = _build_sc_agg(npad, d, bpw)(
        p, q, x_pad, src, dst, z128)
    s_parts = s_parts.reshape(NC, npad, d)
    xs_parts = xs_parts.reshape(NC, npad, d)
    cnt_parts = cnt_parts.reshape(NC, npad, d)

    out_pad = pl.pallas_call(
        _post_body,
        grid=(nb,),
        in_specs=[row_spec,
                  row_spec, row_spec, row_spec, row_spec,
                  row_spec, row_spec,
                  pl.BlockSpec((d, d), lambda i: (0, 0)),
                  pl.BlockSpec((d, 2 * d), lambda i: (0, 0)),
                  pl.BlockSpec((d, d), lambda i: (0, 0))],
        out_specs=row_spec,
        out_shape=jax.ShapeDtypeStruct((npad, d), jnp.float32),
    )(x_pad, s_parts[0], s_parts[1], xs_parts[0], xs_parts[1],
      cnt_parts[0], cnt_parts[1], We1, Wn0, Wn1)

    return out_pad[:n]


# speedup vs baseline: 3.5757x; 1.2393x over previous
import functools

import jax
import jax.numpy as jnp
from jax import lax
from jax.experimental import pallas as pl
from jax.experimental.pallas import tpu as pltpu
from jax.experimental.pallas import tpu_sc as plsc

NC = 2
NS = 16
NW = NC * NS
L = 16
BE = 80
CH = 80
ROWB = 384


def _dot_t(a, w):
    return lax.dot_general(a, w, (((1,), (1,)), ((), ())),
                           preferred_element_type=jnp.float32)


def _pre_body(x_ref, we0_ref, p_ref, q_ref):
    d = x_ref.shape[1]
    a = we0_ref[:, 0:d]
    b = we0_ref[:, d:2 * d]
    c = we0_ref[:, 2 * d:3 * d]
    xb = x_ref[...]
    p_ref[...] = _dot_t(xb, a + c)
    q_ref[...] = _dot_t(xb, b - c)


def _post_body(x_ref, s0_ref, s1_ref, xs0_ref, xs1_ref, c0_ref, c1_ref,
               we1_ref, wn0_ref, wn1_ref, o_ref):
    d = x_ref.shape[1]
    x = x_ref[...]
    s = s0_ref[...] + s1_ref[...]
    xs = xs0_ref[...] + xs1_ref[...]
    cnt = (c0_ref[...] + c1_ref[...])[:, 0:1]
    sums = xs - cnt * x + _dot_t(s, we1_ref[...])
    agg = sums / jnp.maximum(cnt, 1.0)
    u = wn0_ref[:, 0:d]
    v = wn0_ref[:, d:2 * d]
    z = _dot_t(x, u) + _dot_t(agg, v)
    t = jnp.where(z > 0, z, jnp.exp(z) - 1.0)
    o_ref[...] = x + _dot_t(t, wn1_ref[...])


@functools.lru_cache(maxsize=None)
def _build_sc_agg(npad: int, d: int, bpw: int):
    nch = npad // CH
    kmax = -(-nch // NS)
    assert bpw % 2 == 0
    mesh = plsc.VectorSubcoreMesh(core_axis_name="c", subcore_axis_name="s",
                                  num_cores=NC, num_subcores=NS)

    @functools.partial(
        pl.kernel,
        out_type=(
            jax.ShapeDtypeStruct((NC * npad, d), jnp.float32),
            jax.ShapeDtypeStruct((NC * npad, d), jnp.float32),
            jax.ShapeDtypeStruct((NC * npad, d), jnp.float32),
        ),
        mesh=mesh,
        scratch_types=[
            pltpu.VMEM((BE,), jnp.int32),
            pltpu.VMEM((BE,), jnp.int32),
            pltpu.VMEM((BE,), jnp.int32),
            pltpu.VMEM((BE,), jnp.int32),
            pltpu.VMEM((BE, d), jnp.float32),
            pltpu.VMEM((BE, d), jnp.float32),
            pltpu.VMEM((BE, d), jnp.float32),
            pltpu.VMEM((BE, d), jnp.float32),
            pltpu.VMEM_SHARED((npad, d), jnp.float32),
            pltpu.SemaphoreType.DMA,
            pltpu.SemaphoreType.DMA,
            pltpu.SemaphoreType.DMA,
            pltpu.SemaphoreType.DMA,
        ],
    )
    def sc_agg(p_hbm, q_hbm, xp_hbm, src_hbm, dst_hbm, z_hbm,
               s_out, xs_out, cnt_out,
               is0, is1, id0, id1, ra0, ra1, rb0, rb1, acc_sh,
               si0, si1, sg0, sg1):
        cid = lax.axis_index("c")
        sid = lax.axis_index("s")
        wid = sid * NC + cid
        idx_s = (is0, is1)
        idx_d = (id0, id1)
        ra = (ra0, ra1)
        rb = (rb0, rb1)
        si = (si0, si1)
        sg = (sg0, sg1)

        def for_chunks(fn):
            def body(k, _):
                c = jnp.minimum(sid + NS * k, nch - 1)
                fn(c * CH)
                return 0
            lax.fori_loop(0, kmax, body, 0)

        def zero_chunk(off):
            sl = pl.ds(off, CH)
            pltpu.sync_copy(z_hbm.at[sl], acc_sh.at[sl])

        def dump_chunk_to(out_ref):
            def fn(off):
                pltpu.sync_copy(acc_sh.at[pl.ds(off, CH)],
                                out_ref.at[pl.ds(cid * npad + off, CH)])
            return fn

        def fire_idx(k, bi, want_src):
            off = (wid * bpw + bi) * BE
            if want_src:
                pltpu.async_copy(src_hbm.at[pl.ds(off, BE)], idx_s[k], si[k])
            pltpu.async_copy(dst_hbm.at[pl.ds(off, BE)], idx_d[k], si[k])

        def drain_idx(k, want_src):
            if want_src:
                pltpu.make_async_copy(
                    src_hbm.at[pl.ds(0, BE)], idx_s[k], si[k]).wait()
            pltpu.make_async_copy(
                dst_hbm.at[pl.ds(0, BE)], idx_d[k], si[k]).wait()

        def fire_gather(k, tbls):
            bufs = (ra[k], rb[k])
            for t, (tbl, ik) in enumerate(tbls):
                pltpu.async_copy(tbl.at[ik[k]], bufs[t], sg[k])

        def drain_gather(k, tbls):
            bufs = (ra[k], rb[k])
            for t, (tbl, ik) in enumerate(tbls):
                pltpu.make_async_copy(tbl.at[ik[k]], bufs[t], sg[k]).wait()

        def run_phase(tbls, compute, scatter_src, want_src):
            fire_idx(0, 0, want_src)
            drain_idx(0, want_src)
            fire_gather(0, tbls)
            fire_idx(1, 1, want_src)

            def substep(k, bi_next):
                drain_gather(k, tbls)
                drain_idx(1 - k, want_src)
                fire_gather(1 - k, tbls)
                compute(k)
                pltpu.sync_copy(scatter_src(k), acc_sh.at[idx_d[k]],
                                add=True)
                fire_idx(k, bi_next, want_src)

            def body(j, _):
                substep(0, jnp.minimum(2 * j + 2, bpw - 1))
                substep(1, jnp.minimum(2 * j + 3, bpw - 1))
                return 0
            lax.fori_loop(0, bpw // 2, body, 0)
            drain_gather(0, tbls)
            drain_idx(1, want_src)

        for_chunks(zero_chunk)
        plsc.subcore_barrier()

        def elu(k):
            def elu_row(r, _):
                for cc in range(d // L):
                    sl = pl.ds(cc * L, L)
                    z = ra[k][r, sl] + rb[k][r, sl]
                    ra[k][r, sl] = jnp.where(z > 0, z, jnp.exp(z) - 1.0)
                return 0
            lax.fori_loop(0, BE, elu_row, 0)

        run_phase(((p_hbm, idx_s), (q_hbm, idx_d)), elu,
                  lambda k: ra[k], True)
        plsc.subcore_barrier()
        for_chunks(dump_chunk_to(s_out))
        plsc.subcore_barrier()

        for_chunks(zero_chunk)
        plsc.subcore_barrier()
        run_phase(((xp_hbm, idx_s),), lambda k: None,
                  lambda k: ra[k], True)
        plsc.subcore_barrier()
        for_chunks(dump_chunk_to(xs_out))
        plsc.subcore_barrier()

        for_chunks(zero_chunk)

        def fill_ones_row(i, _):
            for cc in range(d // L):
                ra0[i, pl.ds(cc * L, L)] = jnp.ones((L,), jnp.float32)
            return 0
        lax.fori_loop(0, BE, fill_ones_row, 0)
        plsc.subcore_barrier()

        run_phase((), lambda k: None, lambda k: ra0, False)
        plsc.subcore_barrier()
        for_chunks(dump_chunk_to(cnt_out))

    return sc_agg


def kernel(x, edge_index, We0, We1, Wn0, Wn1):
    n, d = x.shape
    e = edge_index.shape[1]
    npad_sc = (n // CH + 1) * CH
    npad_tc = (n // ROWB + 1) * ROWB
    bpw = -(-e // (NW * BE))
    bpw += bpw % 2
    epad = bpw * NW * BE

    x_pad = jnp.pad(x, ((0, npad_tc - n), (0, 0)))
    pad = epad - e
    src = jnp.concatenate([edge_index[0], jnp.full((pad,), n, jnp.int32)])
    dst = jnp.concatenate([edge_index[1], jnp.full((pad,), n, jnp.int32)])

    nb = npad_tc // ROWB
    row_spec = pl.BlockSpec((ROWB, d), lambda i: (i, 0))

    p, q = pl.pallas_call(
        _pre_body,
        grid=(nb,),
        in_specs=[row_spec, pl.BlockSpec((d, 3 * d), lambda i: (0, 0))],
        out_specs=[row_spec, row_spec],
        out_shape=[jax.ShapeDtypeStruct((npad_tc, d), jnp.float32)] * 2,
    )(x_pad, We0)

    z128 = jnp.zeros((npad_sc, d), jnp.float32)
    s_parts, xs_parts, cnt_parts = _build_sc_agg(npad_sc, d, bpw)(
        p, q, x_pad, src, dst, z128)
    tc_pad = ((0, 0), (0, npad_tc - npad_sc), (0, 0))
    s_parts = jnp.pad(s_parts.reshape(NC, npad_sc, d), tc_pad)
    xs_parts = jnp.pad(xs_parts.reshape(NC, npad_sc, d), tc_pad)
    cnt_parts = jnp.pad(cnt_parts.reshape(NC, npad_sc, d), tc_pad)

    out_pad = pl.pallas_call(
        _post_body,
        grid=(nb,),
        in_specs=[row_spec,
                  row_spec, row_spec, row_spec, row_spec,
                  row_spec, row_spec,
                  pl.BlockSpec((d, d), lambda i: (0, 0)),
                  pl.BlockSpec((d, 2 * d), lambda i: (0, 0)),
                  pl.BlockSpec((d, d), lambda i: (0, 0))],
        out_specs=row_spec,
        out_shape=jax.ShapeDtypeStruct((npad_tc, d), jnp.float32),
    )(x_pad, s_parts[0], s_parts[1], xs_parts[0], xs_parts[1],
      cnt_parts[0], cnt_parts[1], We1, Wn0, Wn1)

    return out_pad[:n]


# speedup vs baseline: 3.6377x; 1.0174x over previous
import functools

import jax
import jax.numpy as jnp
from jax import lax
from jax.experimental import pallas as pl
from jax.experimental.pallas import tpu as pltpu
from jax.experimental.pallas import tpu_sc as plsc

NC = 2
NS = 16
NW = NC * NS
L = 16
BE = 80
CH = 80
ROWB = 384


def _dot_t(a, w):
    return lax.dot_general(a, w, (((1,), (1,)), ((), ())),
                           preferred_element_type=jnp.float32)


def _pre_body(x_ref, we0_ref, p_ref, q_ref):
    d = x_ref.shape[1]
    a = we0_ref[:, 0:d]
    b = we0_ref[:, d:2 * d]
    c = we0_ref[:, 2 * d:3 * d]
    xb = x_ref[...]
    p_ref[...] = _dot_t(xb, a + c)
    q_ref[...] = _dot_t(xb, b - c)


def _post_body(x_ref, s0_ref, s1_ref, xs0_ref, xs1_ref, c0_ref, c1_ref,
               we1_ref, wn0_ref, wn1_ref, o_ref):
    d = x_ref.shape[1]
    x = x_ref[...]
    s = s0_ref[...] + s1_ref[...]
    xs = xs0_ref[...] + xs1_ref[...]
    cnt = (c0_ref[...] + c1_ref[...])[:, 0:1]
    sums = xs - cnt * x + _dot_t(s, we1_ref[...])
    agg = sums / jnp.maximum(cnt, 1.0)
    u = wn0_ref[:, 0:d]
    v = wn0_ref[:, d:2 * d]
    z = _dot_t(x, u) + _dot_t(agg, v)
    t = jnp.where(z > 0, z, jnp.exp(z) - 1.0)
    o_ref[...] = x + _dot_t(t, wn1_ref[...])


@functools.lru_cache(maxsize=None)
def _build_sc_agg(npad: int, d: int, bpw: int):
    nch = npad // CH
    kmax = -(-nch // NS)
    assert bpw % 2 == 0
    mesh = plsc.VectorSubcoreMesh(core_axis_name="c", subcore_axis_name="s",
                                  num_cores=NC, num_subcores=NS)

    @functools.partial(
        pl.kernel,
        out_type=(
            jax.ShapeDtypeStruct((NC * npad, d), jnp.float32),
            jax.ShapeDtypeStruct((NC * npad, d), jnp.float32),
            jax.ShapeDtypeStruct((NC * npad, d), jnp.float32),
        ),
        mesh=mesh,
        scratch_types=[
            pltpu.VMEM((BE,), jnp.int32),
            pltpu.VMEM((BE,), jnp.int32),
            pltpu.VMEM((BE,), jnp.int32),
            pltpu.VMEM((BE,), jnp.int32),
            pltpu.VMEM((BE, d), jnp.float32),
            pltpu.VMEM((BE, d), jnp.float32),
            pltpu.VMEM((BE, d), jnp.float32),
            pltpu.VMEM((BE, d), jnp.float32),
            pltpu.VMEM_SHARED((npad, d), jnp.float32),
            pltpu.SemaphoreType.DMA,
            pltpu.SemaphoreType.DMA,
            pltpu.SemaphoreType.DMA,
            pltpu.SemaphoreType.DMA,
        ],
    )
    def sc_agg(p_hbm, q_hbm, xp_hbm, src_hbm, dst_hbm, z_hbm,
               s_out, xs_out, cnt_out,
               is0, is1, id0, id1, ra0, ra1, rb0, rb1, acc_sh,
               si0, si1, sg0, sg1):
        cid = lax.axis_index("c")
        sid = lax.axis_index("s")
        wid = sid * NC + cid
        idx_s = (is0, is1)
        idx_d = (id0, id1)
        ra = (ra0, ra1)
        rb = (rb0, rb1)
        si = (si0, si1)
        sg = (sg0, sg1)

        def for_chunks(fn):
            def body(k, _):
                c = jnp.minimum(sid + NS * k, nch - 1)
                fn(c * CH)
                return 0
            lax.fori_loop(0, kmax, body, 0)

        def zero_chunk(off):
            sl = pl.ds(off, CH)
            pltpu.sync_copy(z_hbm.at[sl], acc_sh.at[sl])

        def dump_chunk_to(out_ref):
            def fn(off):
                pltpu.sync_copy(acc_sh.at[pl.ds(off, CH)],
                                out_ref.at[pl.ds(cid * npad + off, CH)])
            return fn

        def fire_idx(k, bi, want_src):
            off = (wid * bpw + bi) * BE
            if want_src:
                pltpu.async_copy(src_hbm.at[pl.ds(off, BE)], idx_s[k], si[k])
            pltpu.async_copy(dst_hbm.at[pl.ds(off, BE)], idx_d[k], si[k])

        def drain_idx(k, want_src):
            if want_src:
                pltpu.make_async_copy(
                    src_hbm.at[pl.ds(0, BE)], idx_s[k], si[k]).wait()
            pltpu.make_async_copy(
                dst_hbm.at[pl.ds(0, BE)], idx_d[k], si[k]).wait()

        def fire_gather(k, tbls):
            bufs = (ra[k], rb[k])
            for t, (tbl, ik) in enumerate(tbls):
                pltpu.async_copy(tbl.at[ik[k]], bufs[t], sg[k])

        def drain_gather(k, tbls):
            bufs = (ra[k], rb[k])
            for t, (tbl, ik) in enumerate(tbls):
                pltpu.make_async_copy(tbl.at[ik[k]], bufs[t], sg[k]).wait()

        def run_phase(tbls, compute, scatter_src, want_src):
            fire_idx(0, 0, want_src)
            drain_idx(0, want_src)
            fire_gather(0, tbls)
            fire_idx(1, 1, want_src)

            def substep(k, bi_next):
                drain_gather(k, tbls)
                drain_idx(1 - k, want_src)
                fire_gather(1 - k, tbls)
                compute(k)
                pltpu.sync_copy(scatter_src(k), acc_sh.at[idx_d[k]],
                                add=True)
                fire_idx(k, bi_next, want_src)

            def body(j, _):
                substep(0, jnp.minimum(2 * j + 2, bpw - 1))
                substep(1, jnp.minimum(2 * j + 3, bpw - 1))
                return 0
            lax.fori_loop(0, bpw // 2, body, 0)
            drain_gather(0, tbls)
            drain_idx(1, want_src)

        for_chunks(zero_chunk)
        plsc.subcore_barrier()

        def elu(k):
            @plsc.parallel_loop(0, BE, unroll=4)
            def _(r):
                for cc in range(d // L):
                    sl = pl.ds(cc * L, L)
                    z = ra[k][r, sl] + rb[k][r, sl]
                    ra[k][r, sl] = jnp.where(z > 0, z, jnp.exp(z) - 1.0)

        run_phase(((p_hbm, idx_s), (q_hbm, idx_d)), elu,
                  lambda k: ra[k], True)
        plsc.subcore_barrier()
        for_chunks(dump_chunk_to(s_out))
        plsc.subcore_barrier()

        for_chunks(zero_chunk)
        plsc.subcore_barrier()
        run_phase(((xp_hbm, idx_s),), lambda k: None,
                  lambda k: ra[k], True)
        plsc.subcore_barrier()
        for_chunks(dump_chunk_to(xs_out))
        plsc.subcore_barrier()

        for_chunks(zero_chunk)

        def fill_ones_row(i, _):
            for cc in range(d // L):
                ra0[i, pl.ds(cc * L, L)] = jnp.ones((L,), jnp.float32)
            return 0
        lax.fori_loop(0, BE, fill_ones_row, 0)
        plsc.subcore_barrier()

        run_phase((), lambda k: None, lambda k: ra0, False)
        plsc.subcore_barrier()
        for_chunks(dump_chunk_to(cnt_out))

    return sc_agg


def kernel(x, edge_index, We0, We1, Wn0, Wn1):
    n, d = x.shape
    e = edge_index.shape[1]
    npad_sc = (n // CH + 1) * CH
    npad_tc = (n // ROWB + 1) * ROWB
    bpw = -(-e // (NW * BE))
    bpw += bpw % 2
    epad = bpw * NW * BE

    x_pad = jnp.pad(x, ((0, npad_tc - n), (0, 0)))
    pad = epad - e
    src = jnp.concatenate([edge_index[0], jnp.full((pad,), n, jnp.int32)])
    dst = jnp.concatenate([edge_index[1], jnp.full((pad,), n, jnp.int32)])

    nb = npad_tc // ROWB
    row_spec = pl.BlockSpec((ROWB, d), lambda i: (i, 0))

    p, q = pl.pallas_call(
        _pre_body,
        grid=(nb,),
        in_specs=[row_spec, pl.BlockSpec((d, 3 * d), lambda i: (0, 0))],
        out_specs=[row_spec, row_spec],
        out_shape=[jax.ShapeDtypeStruct((npad_tc, d), jnp.float32)] * 2,
    )(x_pad, We0)

    z128 = jnp.zeros((npad_sc, d), jnp.float32)
    s_parts, xs_parts, cnt_parts = _build_sc_agg(npad_sc, d, bpw)(
        p, q, x_pad, src, dst, z128)
    tc_pad = ((0, 0), (0, npad_tc - npad_sc), (0, 0))
    s_parts = jnp.pad(s_parts.reshape(NC, npad_sc, d), tc_pad)
    xs_parts = jnp.pad(xs_parts.reshape(NC, npad_sc, d), tc_pad)
    cnt_parts = jnp.pad(cnt_parts.reshape(NC, npad_sc, d), tc_pad)

    out_pad = pl.pallas_call(
        _post_body,
        grid=(nb,),
        in_specs=[row_spec,
                  row_spec, row_spec, row_spec, row_spec,
                  row_spec, row_spec,
                  pl.BlockSpec((d, d), lambda i: (0, 0)),
                  pl.BlockSpec((d, 2 * d), lambda i: (0, 0)),
                  pl.BlockSpec((d, d), lambda i: (0, 0))],
        out_specs=row_spec,
        out_shape=jax.ShapeDtypeStruct((npad_tc, d), jnp.float32),
    )(x_pad, s_parts[0], s_parts[1], xs_parts[0], xs_parts[1],
      cnt_parts[0], cnt_parts[1], We1, Wn0, Wn1)

    return out_pad[:n]


# speedup vs baseline: 3.6386x; 1.0002x over previous
import functools

import jax
import jax.numpy as jnp
from jax import lax
from jax.experimental import pallas as pl
from jax.experimental.pallas import tpu as pltpu
from jax.experimental.pallas import tpu_sc as plsc

NC = 2
NS = 16
NW = NC * NS
L = 16
BE = 80
CH = 80
ROWB = 384


def _dot_t(a, w):
    return lax.dot_general(a, w, (((1,), (1,)), ((), ())),
                           preferred_element_type=jnp.float32)


def _pre_body(x_ref, we0_ref, p_ref, q_ref):
    d = x_ref.shape[1]
    a = we0_ref[:, 0:d]
    b = we0_ref[:, d:2 * d]
    c = we0_ref[:, 2 * d:3 * d]
    xb = x_ref[...]
    p_ref[...] = _dot_t(xb, a + c)
    q_ref[...] = _dot_t(xb, b - c)


def _post_body(x_ref, s0_ref, s1_ref, xs0_ref, xs1_ref, c0_ref, c1_ref,
               we1_ref, wn0_ref, wn1_ref, o_ref):
    d = x_ref.shape[1]
    x = x_ref[...]
    s = s0_ref[...] + s1_ref[...]
    xs = xs0_ref[...] + xs1_ref[...]
    cnt = (c0_ref[...] + c1_ref[...])[:, 0:1]
    sums = xs - cnt * x + _dot_t(s, we1_ref[...])
    agg = sums / jnp.maximum(cnt, 1.0)
    u = wn0_ref[:, 0:d]
    v = wn0_ref[:, d:2 * d]
    z = _dot_t(x, u) + _dot_t(agg, v)
    t = jnp.where(z > 0, z, jnp.exp(z) - 1.0)
    o_ref[...] = x + _dot_t(t, wn1_ref[...])


@functools.lru_cache(maxsize=None)
def _build_sc_agg(npad: int, d: int, bpw: int):
    nch = npad // CH
    kmax = -(-nch // NS)
    assert bpw % 2 == 0
    mesh = plsc.VectorSubcoreMesh(core_axis_name="c", subcore_axis_name="s",
                                  num_cores=NC, num_subcores=NS)

    @functools.partial(
        pl.kernel,
        out_type=(
            jax.ShapeDtypeStruct((NC * npad, d), jnp.float32),
            jax.ShapeDtypeStruct((NC * npad, d), jnp.float32),
            jax.ShapeDtypeStruct((NC * npad, d), jnp.float32),
        ),
        mesh=mesh,
        scratch_types=[
            pltpu.VMEM((BE,), jnp.int32),
            pltpu.VMEM((BE,), jnp.int32),
            pltpu.VMEM((BE,), jnp.int32),
            pltpu.VMEM((BE,), jnp.int32),
            pltpu.VMEM((BE, d), jnp.float32),
            pltpu.VMEM((BE, d), jnp.float32),
            pltpu.VMEM((BE, d), jnp.float32),
            pltpu.VMEM((BE, d), jnp.float32),
            pltpu.VMEM_SHARED((npad, d), jnp.float32),
            pltpu.SemaphoreType.DMA,
            pltpu.SemaphoreType.DMA,
            pltpu.SemaphoreType.DMA,
            pltpu.SemaphoreType.DMA,
        ],
    )
    def sc_agg(p_hbm, q_hbm, xp_hbm, src_hbm, dst_hbm, z_hbm,
               s_out, xs_out, cnt_out,
               is0, is1, id0, id1, ra0, ra1, rb0, rb1, acc_sh,
               si0, si1, sg0, sg1):
        cid = lax.axis_index("c")
        sid = lax.axis_index("s")
        wid = sid * NC + cid
        idx_s = (is0, is1)
        idx_d = (id0, id1)
        ra = (ra0, ra1)
        rb = (rb0, rb1)
        si = (si0, si1)
        sg = (sg0, sg1)

        def for_chunks(fn):
            def body(k, _):
                c = jnp.minimum(sid + NS * k, nch - 1)
                fn(c * CH)
                return 0
            lax.fori_loop(0, kmax, body, 0)

        def zero_chunk(off):
            sl = pl.ds(off, CH)
            pltpu.sync_copy(z_hbm.at[sl], acc_sh.at[sl])

        def dump_chunk_to(out_ref):
            def fn(off):
                pltpu.sync_copy(acc_sh.at[pl.ds(off, CH)],
                                out_ref.at[pl.ds(cid * npad + off, CH)])
            return fn

        def fire_idx(k, bi, want_src):
            off = (wid * bpw + bi) * BE
            if want_src:
                pltpu.async_copy(src_hbm.at[pl.ds(off, BE)], idx_s[k], si[k])
            pltpu.async_copy(dst_hbm.at[pl.ds(off, BE)], idx_d[k], si[k])

        def drain_idx(k, want_src):
            if want_src:
                pltpu.make_async_copy(
                    src_hbm.at[pl.ds(0, BE)], idx_s[k], si[k]).wait()
            pltpu.make_async_copy(
                dst_hbm.at[pl.ds(0, BE)], idx_d[k], si[k]).wait()

        def fire_gather(k, tbls):
            bufs = (ra[k], rb[k])
            for t, (tbl, ik) in enumerate(tbls):
                pltpu.async_copy(tbl.at[ik[k]], bufs[t], sg[k])

        def drain_gather(k, tbls):
            bufs = (ra[k], rb[k])
            for t, (tbl, ik) in enumerate(tbls):
                pltpu.make_async_copy(tbl.at[ik[k]], bufs[t], sg[k]).wait()

        def run_phase(tbls, compute, scatter_src, want_src):
            fire_idx(0, 0, want_src)
            drain_idx(0, want_src)
            fire_gather(0, tbls)
            fire_idx(1, 1, want_src)

            def substep(k, bi_next):
                drain_gather(k, tbls)
                drain_idx(1 - k, want_src)
                fire_gather(1 - k, tbls)
                compute(k)
                pltpu.sync_copy(scatter_src(k), acc_sh.at[idx_d[k]],
                                add=True)
                fire_idx(k, bi_next, want_src)

            def body(j, _):
                substep(0, jnp.minimum(2 * j + 2, bpw - 1))
                substep(1, jnp.minimum(2 * j + 3, bpw - 1))
                return 0
            lax.fori_loop(0, bpw // 2, body, 0)
            drain_gather(0, tbls)
            drain_idx(1, want_src)

        for_chunks(zero_chunk)
        plsc.subcore_barrier()

        def elu(k):
            @plsc.parallel_loop(0, BE, unroll=8)
            def _(r):
                for cc in range(d // L):
                    sl = pl.ds(cc * L, L)
                    z = ra[k][r, sl] + rb[k][r, sl]
                    ra[k][r, sl] = jnp.where(z > 0, z, jnp.exp(z) - 1.0)

        run_phase(((p_hbm, idx_s), (q_hbm, idx_d)), elu,
                  lambda k: ra[k], True)
        plsc.subcore_barrier()
        for_chunks(dump_chunk_to(s_out))
        plsc.subcore_barrier()

        for_chunks(zero_chunk)
        plsc.subcore_barrier()
        run_phase(((xp_hbm, idx_s),), lambda k: None,
                  lambda k: ra[k], True)
        plsc.subcore_barrier()
        for_chunks(dump_chunk_to(xs_out))
        plsc.subcore_barrier()

        for_chunks(zero_chunk)

        def fill_ones_row(i, _):
            for cc in range(d // L):
                ra0[i, pl.ds(cc * L, L)] = jnp.ones((L,), jnp.float32)
            return 0
        lax.fori_loop(0, BE, fill_ones_row, 0)
        plsc.subcore_barrier()

        run_phase((), lambda k: None, lambda k: ra0, False)
        plsc.subcore_barrier()
        for_chunks(dump_chunk_to(cnt_out))

    return sc_agg


def kernel(x, edge_index, We0, We1, Wn0, Wn1):
    n, d = x.shape
    e = edge_index.shape[1]
    npad_sc = (n // CH + 1) * CH
    npad_tc = (n // ROWB + 1) * ROWB
    bpw = -(-e // (NW * BE))
    bpw += bpw % 2
    epad = bpw * NW * BE

    x_pad = jnp.pad(x, ((0, npad_tc - n), (0, 0)))
    pad = epad - e
    src = jnp.concatenate([edge_index[0], jnp.full((pad,), n, jnp.int32)])
    dst = jnp.concatenate([edge_index[1], jnp.full((pad,), n, jnp.int32)])

    nb = npad_tc // ROWB
    row_spec = pl.BlockSpec((ROWB, d), lambda i: (i, 0))

    p, q = pl.pallas_call(
        _pre_body,
        grid=(nb,),
        in_specs=[row_spec, pl.BlockSpec((d, 3 * d), lambda i: (0, 0))],
        out_specs=[row_spec, row_spec],
        out_shape=[jax.ShapeDtypeStruct((npad_tc, d), jnp.float32)] * 2,
    )(x_pad, We0)

    z128 = jnp.zeros((npad_sc, d), jnp.float32)
    s_parts, xs_parts, cnt_parts = _build_sc_agg(npad_sc, d, bpw)(
        p, q, x_pad, src, dst, z128)
    tc_pad = ((0, 0), (0, npad_tc - npad_sc), (0, 0))
    s_parts = jnp.pad(s_parts.reshape(NC, npad_sc, d), tc_pad)
    xs_parts = jnp.pad(xs_parts.reshape(NC, npad_sc, d), tc_pad)
    cnt_parts = jnp.pad(cnt_parts.reshape(NC, npad_sc, d), tc_pad)

    out_pad = pl.pallas_call(
        _post_body,
        grid=(nb,),
        in_specs=[row_spec,
                  row_spec, row_spec, row_spec, row_spec,
                  row_spec, row_spec,
                  pl.BlockSpec((d, d), lambda i: (0, 0)),
                  pl.BlockSpec((d, 2 * d), lambda i: (0, 0)),
                  pl.BlockSpec((d, d), lambda i: (0, 0))],
        out_specs=row_spec,
        out_shape=jax.ShapeDtypeStruct((npad_tc, d), jnp.float32),
    )(x_pad, s_parts[0], s_parts[1], xs_parts[0], xs_parts[1],
      cnt_parts[0], cnt_parts[1], We1, Wn0, Wn1)

    return out_pad[:n]


# speedup vs baseline: 3.6471x; 1.0023x over previous
import functools

import jax
import jax.numpy as jnp
from jax import lax
from jax.experimental import pallas as pl
from jax.experimental.pallas import tpu as pltpu
from jax.experimental.pallas import tpu_sc as plsc

NC = 2
NS = 16
NW = NC * NS
L = 16
BE = 80
CH = 80
ROWB = 384


def _dot_t(a, w):
    return lax.dot_general(a, w, (((1,), (1,)), ((), ())),
                           preferred_element_type=jnp.float32)


def _pre_body(x_ref, we0_ref, p_ref, q_ref):
    d = x_ref.shape[1]
    a = we0_ref[:, 0:d]
    b = we0_ref[:, d:2 * d]
    c = we0_ref[:, 2 * d:3 * d]
    xb = x_ref[...]
    p_ref[...] = _dot_t(xb, a + c)
    q_ref[...] = _dot_t(xb, b - c)


def _post_body(x_ref, s0_ref, s1_ref, xs0_ref, xs1_ref, c0_ref, c1_ref,
               we1_ref, wn0_ref, wn1_ref, o_ref):
    d = x_ref.shape[1]
    x = x_ref[...]
    s = s0_ref[...] + s1_ref[...]
    xs = xs0_ref[...] + xs1_ref[...]
    cnt = (c0_ref[...] + c1_ref[...])[:, 0:1]
    sums = xs - cnt * x + _dot_t(s, we1_ref[...])
    agg = sums / jnp.maximum(cnt, 1.0)
    u = wn0_ref[:, 0:d]
    v = wn0_ref[:, d:2 * d]
    z = _dot_t(x, u) + _dot_t(agg, v)
    t = jnp.where(z > 0, z, jnp.exp(z) - 1.0)
    o_ref[...] = x + _dot_t(t, wn1_ref[...])


@functools.lru_cache(maxsize=None)
def _build_sc_agg(npad: int, d: int, bpw: int):
    nch = npad // CH
    kmax = nch // NS
    assert nch % NS == 0 and bpw % 2 == 0
    mesh = plsc.VectorSubcoreMesh(core_axis_name="c", subcore_axis_name="s",
                                  num_cores=NC, num_subcores=NS)

    @functools.partial(
        pl.kernel,
        out_type=(
            jax.ShapeDtypeStruct((NC * npad, d), jnp.float32),
            jax.ShapeDtypeStruct((NC * npad, d), jnp.float32),
            jax.ShapeDtypeStruct((NC * npad, d), jnp.float32),
        ),
        mesh=mesh,
        scratch_types=[
            pltpu.VMEM((BE,), jnp.int32),
            pltpu.VMEM((BE,), jnp.int32),
            pltpu.VMEM((BE,), jnp.int32),
            pltpu.VMEM((BE,), jnp.int32),
            pltpu.VMEM((BE, d), jnp.float32),
            pltpu.VMEM((BE, d), jnp.float32),
            pltpu.VMEM((BE, d), jnp.float32),
            pltpu.VMEM((BE, d), jnp.float32),
            pltpu.VMEM_SHARED((npad, d), jnp.float32),
            pltpu.SemaphoreType.DMA,
            pltpu.SemaphoreType.DMA,
            pltpu.SemaphoreType.DMA,
            pltpu.SemaphoreType.DMA,
        ],
    )
    def sc_agg(p_hbm, q_hbm, xp_hbm, src_hbm, dst_hbm, z_hbm,
               s_out, xs_out, cnt_out,
               is0, is1, id0, id1, ra0, ra1, rb0, rb1, acc_sh,
               si0, si1, sg0, sg1):
        cid = lax.axis_index("c")
        sid = lax.axis_index("s")
        wid = sid * NC + cid
        idx_s = (is0, is1)
        idx_d = (id0, id1)
        ra = (ra0, ra1)
        rb = (rb0, rb1)
        si = (si0, si1)
        sg = (sg0, sg1)

        def for_chunks(fn):
            def body(k, _):
                fn((sid + NS * k) * CH)
                return 0
            lax.fori_loop(0, kmax, body, 0)

        def zero_chunk(off):
            sl = pl.ds(off, CH)
            pltpu.sync_copy(z_hbm.at[sl], acc_sh.at[sl])

        def dump_chunk_to(out_ref, rezero):
            def fn(off):
                sl = pl.ds(off, CH)
                pltpu.sync_copy(acc_sh.at[sl],
                                out_ref.at[pl.ds(cid * npad + off, CH)])
                if rezero:
                    pltpu.sync_copy(z_hbm.at[sl], acc_sh.at[sl])
            return fn

        def fire_idx(k, bi, want_src):
            off = (wid * bpw + bi) * BE
            if want_src:
                pltpu.async_copy(src_hbm.at[pl.ds(off, BE)], idx_s[k], si[k])
            pltpu.async_copy(dst_hbm.at[pl.ds(off, BE)], idx_d[k], si[k])

        def drain_idx(k, want_src):
            if want_src:
                pltpu.make_async_copy(
                    src_hbm.at[pl.ds(0, BE)], idx_s[k], si[k]).wait()
            pltpu.make_async_copy(
                dst_hbm.at[pl.ds(0, BE)], idx_d[k], si[k]).wait()

        def fire_gather(k, tbls):
            bufs = (ra[k], rb[k])
            for t, (tbl, ik) in enumerate(tbls):
                pltpu.async_copy(tbl.at[ik[k]], bufs[t], sg[k])

        def drain_gather(k, tbls):
            bufs = (ra[k], rb[k])
            for t, (tbl, ik) in enumerate(tbls):
                pltpu.make_async_copy(tbl.at[ik[k]], bufs[t], sg[k]).wait()

        def run_phase(tbls, compute, scatter_src, want_src):
            fire_idx(0, 0, want_src)
            drain_idx(0, want_src)
            fire_gather(0, tbls)
            fire_idx(1, 1, want_src)

            def substep(k, bi_next):
                drain_gather(k, tbls)
                drain_idx(1 - k, want_src)
                fire_gather(1 - k, tbls)
                compute(k)
                pltpu.sync_copy(scatter_src(k), acc_sh.at[idx_d[k]],
                                add=True)
                fire_idx(k, bi_next, want_src)

            def body(j, _):
                substep(0, jnp.minimum(2 * j + 2, bpw - 1))
                substep(1, jnp.minimum(2 * j + 3, bpw - 1))
                return 0
            lax.fori_loop(0, bpw // 2, body, 0)
            drain_gather(0, tbls)
            drain_idx(1, want_src)

        for_chunks(zero_chunk)
        plsc.subcore_barrier()

        def elu(k):
            @plsc.parallel_loop(0, BE, unroll=8)
            def _(r):
                for cc in range(d // L):
                    sl = pl.ds(cc * L, L)
                    z = ra[k][r, sl] + rb[k][r, sl]
                    ra[k][r, sl] = jnp.where(z > 0, z, jnp.exp(z) - 1.0)

        run_phase(((p_hbm, idx_s), (q_hbm, idx_d)), elu,
                  lambda k: ra[k], True)
        plsc.subcore_barrier()
        for_chunks(dump_chunk_to(s_out, rezero=True))
        plsc.subcore_barrier()

        run_phase(((xp_hbm, idx_s),), lambda k: None,
                  lambda k: ra[k], True)
        plsc.subcore_barrier()
        for_chunks(dump_chunk_to(xs_out, rezero=True))

        def fill_ones_row(i, _):
            for cc in range(d // L):
                ra0[i, pl.ds(cc * L, L)] = jnp.ones((L,), jnp.float32)
            return 0
        lax.fori_loop(0, BE, fill_ones_row, 0)
        plsc.subcore_barrier()

        run_phase((), lambda k: None, lambda k: ra0, False)
        plsc.subcore_barrier()
        for_chunks(dump_chunk_to(cnt_out, rezero=False))

    return sc_agg


def kernel(x, edge_index, We0, We1, Wn0, Wn1):
    n, d = x.shape
    e = edge_index.shape[1]
    npad_sc = (n // (CH * NS) + 1) * (CH * NS)
    npad_tc = (n // ROWB + 1) * ROWB
    bpw = -(-e // (NW * BE))
    bpw += bpw % 2
    epad = bpw * NW * BE

    x_pad = jnp.pad(x, ((0, npad_tc - n), (0, 0)))
    pad = epad - e
    src = jnp.concatenate([edge_index[0], jnp.full((pad,), n, jnp.int32)])
    dst = jnp.concatenate([edge_index[1], jnp.full((pad,), n, jnp.int32)])

    nb = npad_tc // ROWB
    row_spec = pl.BlockSpec((ROWB, d), lambda i: (i, 0))

    p, q = pl.pallas_call(
        _pre_body,
        grid=(nb,),
        in_specs=[row_spec, pl.BlockSpec((d, 3 * d), lambda i: (0, 0))],
        out_specs=[row_spec, row_spec],
        out_shape=[jax.ShapeDtypeStruct((npad_tc, d), jnp.float32)] * 2,
    )(x_pad, We0)

    z128 = jnp.zeros((npad_sc, d), jnp.float32)
    s_parts, xs_parts, cnt_parts = _build_sc_agg(npad_sc, d, bpw)(
        p, q, x_pad, src, dst, z128)
    tc_pad = ((0, 0), (0, npad_tc - npad_sc), (0, 0))
    s_parts = jnp.pad(s_parts.reshape(NC, npad_sc, d), tc_pad)
    xs_parts = jnp.pad(xs_parts.reshape(NC, npad_sc, d), tc_pad)
    cnt_parts = jnp.pad(cnt_parts.reshape(NC, npad_sc, d), tc_pad)

    out_pad = pl.pallas_call(
        _post_body,
        grid=(nb,),
        in_specs=[row_spec,
                  row_spec, row_spec, row_spec, row_spec,
                  row_spec, row_spec,
                  pl.BlockSpec((d, d), lambda i: (0, 0)),
                  pl.BlockSpec((d, 2 * d), lambda i: (0, 0)),
                  pl.BlockSpec((d, d), lambda i: (0, 0))],
        out_specs=row_spec,
        out_shape=jax.ShapeDtypeStruct((npad_tc, d), jnp.float32),
    )(x_pad, s_parts[0], s_parts[1], xs_parts[0], xs_parts[1],
      cnt_parts[0], cnt_parts[1], We1, Wn0, Wn1)

    return out_pad[:n]
